# Initial kernel scaffold; baseline (speedup 1.0000x reference)
import jax
import jax.numpy as jnp
from jax.experimental import pallas as pl


def kernel(x, c1_w, c1_scale, c1_bias, l0b0_c1_w, l0b0_c1_scale, l0b0_c1_bias, l0b0_c2_w, l0b0_c2_scale, l0b0_c2_bias, l0b1_c1_w, l0b1_c1_scale, l0b1_c1_bias, l0b1_c2_w, l0b1_c2_scale, l0b1_c2_bias, l1b0_c1_w, l1b0_c1_scale, l1b0_c1_bias, l1b0_c2_w, l1b0_c2_scale, l1b0_c2_bias, l1b0_dn_w, l1b0_dn_scale, l1b0_dn_bias, l1b1_c1_w, l1b1_c1_scale, l1b1_c1_bias, l1b1_c2_w, l1b1_c2_scale, l1b1_c2_bias, l2b0_c1_w, l2b0_c1_scale, l2b0_c1_bias, l2b0_c2_w, l2b0_c2_scale, l2b0_c2_bias, l2b0_dn_w, l2b0_dn_scale, l2b0_dn_bias, l2b1_c1_w, l2b1_c1_scale, l2b1_c1_bias, l2b1_c2_w, l2b1_c2_scale, l2b1_c2_bias, l3b0_c1_w, l3b0_c1_scale, l3b0_c1_bias, l3b0_c2_w, l3b0_c2_scale, l3b0_c2_bias, l3b0_dn_w, l3b0_dn_scale, l3b0_dn_bias, l3b1_c1_w, l3b1_c1_scale, l3b1_c1_bias, l3b1_c2_w, l3b1_c2_scale, l3b1_c2_bias, hd_w, hd_bias):
    raise NotImplementedError("write your pallas kernel here")


# speedup vs baseline: 1.0273x; 1.0273x over previous
import jax
import jax.numpy as jnp
from jax.experimental import pallas as pl
from jax.experimental.pallas import tpu as pltpu


def _full(i):
    def im(_):
        return (0,) * i
    return im


def _plan_tm(m):
    tm = m
    while tm > 1024 and tm % 2 == 0 and (tm // 2) % 8 == 0:
        tm //= 2
    return tm


def _mm_body(relu, has_res):
    def body(*refs):
        if has_res:
            a_ref, w_ref, s_ref, b_ref, r_ref, o_ref = refs
        else:
            a_ref, w_ref, s_ref, b_ref, o_ref = refs
            r_ref = None
        acc = jnp.dot(a_ref[...], w_ref[...],
                      preferred_element_type=jnp.float32)
        y = acc * s_ref[...] + b_ref[...]
        if has_res:
            y = y + r_ref[...].astype(jnp.float32)
        if relu:
            y = jnp.maximum(y, 0.0)
        o_ref[...] = y.astype(o_ref.dtype)
    return body


def _mm(a, w, s, b, residual=None, relu=False):
    m, k = a.shape
    n = w.shape[1]
    tm = _plan_tm(m)
    nm = m // tm
    has_res = residual is not None
    in_specs = [
        pl.BlockSpec((tm, k), lambda i, j, kk: (i, kk)),
        pl.BlockSpec((k, n), lambda i, j, kk: (kk, j)),
        pl.BlockSpec((1, n), lambda i, j, kk: (0, j)),
        pl.BlockSpec((1, n), lambda i, j, kk: (0, j)),
    ]
    inputs = [a, w, s, b]
    if has_res:
        in_specs.append(pl.BlockSpec((tm, n), lambda i, j, kk: (i, j)))
        inputs.append(residual)
    vmem = (2 * (tm * k + k * n) * 2 + 2 * tm * n * 2
            + (2 * tm * n * 2 if has_res else 0)
            + 4 * n * 4 * 2 + (4 << 20))
    vmem = int(min(max(vmem, 16 << 20), 48 << 20))
    return pl.pallas_call(
        _mm_body(relu, has_res),
        out_shape=jax.ShapeDtypeStruct((m, n), jnp.bfloat16),
        grid_spec=pltpu.PrefetchScalarGridSpec(
            num_scalar_prefetch=0,
            grid=(nm, 1, 1),
            in_specs=in_specs,
            out_specs=pl.BlockSpec((tm, n), lambda i, j, kk: (i, j)),
            scratch_shapes=[],
        ),
        compiler_params=pltpu.CompilerParams(
            dimension_semantics=("parallel", "parallel", "arbitrary"),
            vmem_limit_bytes=vmem),
    )(*inputs)


def _im2col(x2d, n, h, w, c, kh, kw, stride, pad):
    x = x2d.reshape(n, h, w, c)
    ho = (h + 2 * pad - kh) // stride + 1
    wo = (w + 2 * pad - kw) // stride + 1
    if kh == 1 and kw == 1 and pad == 0:
        return x[:, ::stride, ::stride, :].reshape(n * ho * wo, c), ho, wo
    xp = jnp.pad(x, ((0, 0), (pad, pad), (pad, pad), (0, 0)))
    cols = []
    for i in range(kh):
        for j in range(kw):
            cols.append(xp[:, i:i + ho * stride:stride,
                           j:j + wo * stride:stride, :])
    p = jnp.stack(cols, axis=3).reshape(n * ho * wo, kh * kw * c)
    return p, ho, wo


def _conv_bn(x2d, n, h, w, c, wsb, kh, kw, stride, pad, relu, residual=None):
    wt, s, b = wsb
    patches, ho, wo = _im2col(x2d, n, h, w, c, kh, kw, stride, pad)
    y = _mm(patches, wt, s, b, residual=residual, relu=relu)
    return y, ho, wo


def _block(x2d, n, h, c, c1, c2, dn, stride):
    y1, ho, _ = _conv_bn(x2d, n, h, h, c, c1, 3, 3, stride, 1, True)
    if dn is not None:
        ident, _, _ = _conv_bn(x2d, n, h, h, c, dn, 1, 1, stride, 0, False)
    else:
        ident = x2d
    cout = y1.shape[-1]
    y2, _, _ = _conv_bn(y1, n, ho, ho, cout, c2, 3, 3, 1, 1, True,
                        residual=ident)
    return y2, ho, cout


def _stem_kernel(a_ref, w_ref, s_ref, b_ref, o_ref):
    acc = jnp.dot(a_ref[...], w_ref[...], preferred_element_type=jnp.float32)
    y = jnp.maximum(acc * s_ref[...] + b_ref[...], 0.0).astype(jnp.bfloat16)
    ns = o_ref.shape[0] // 256
    y = y.reshape(ns, 32, 32, 128)
    yp = jnp.pad(y, ((0, 0), (1, 1), (1, 1), (0, 0)),
                 constant_values=jnp.array(-jnp.inf, jnp.bfloat16))
    mp = None
    for i in range(3):
        for j in range(3):
            sl = jax.lax.slice(yp, (0, i, j, 0), (ns, i + 32, j + 32, 128))
            sl = sl.reshape(ns, 16, 2, 16, 2, 128)[:, :, 0, :, 0, :]
            mp = sl if mp is None else jnp.maximum(mp, sl)
    o_ref[...] = mp.reshape(ns * 256, 128)


def _stage_a(x, c1_w, c1_scale, c1_bias):
    n = x.shape[0]
    ns = 8 if n % 8 == 0 else n
    xh = jnp.transpose(x, (0, 2, 3, 1)).astype(jnp.bfloat16)
    xp = jnp.pad(xh, ((0, 0), (3, 3), (3, 3), (0, 0)))
    cols = []
    for i in range(7):
        for j in range(7):
            cols.append(xp[:, i:i + 64:2, j:j + 64:2, :])
    p = jnp.stack(cols, axis=3).reshape(n * 1024, 147)
    patches = jnp.pad(p, ((0, 0), (0, 109)))
    return pl.pallas_call(
        _stem_kernel,
        out_shape=jax.ShapeDtypeStruct((n * 256, 128), jnp.bfloat16),
        grid=(n // ns,),
        in_specs=[
            pl.BlockSpec((ns * 1024, 256), lambda i: (i, 0)),
            pl.BlockSpec((256, 128), _full(2)),
            pl.BlockSpec((1, 128), _full(2)),
            pl.BlockSpec((1, 128), _full(2)),
        ],
        out_specs=pl.BlockSpec((ns * 256, 128), lambda i: (i, 0)),
        compiler_params=pltpu.CompilerParams(
            dimension_semantics=("parallel",),
            vmem_limit_bytes=int(48 << 20)),
    )(patches, c1_w, c1_scale, c1_bias)


def _heads_kernel(x_ref, w_ref, b_ref, o_ref):
    nb = o_ref.shape[0]
    xv = x_ref[...].reshape(nb, 4, 512)
    xs = jnp.sum(xv.astype(jnp.float32), axis=1)
    feat = (xs * 0.25).astype(jnp.bfloat16)
    o_ref[...] = (jnp.dot(feat, w_ref[...],
                          preferred_element_type=jnp.float32) + b_ref[...])


def _stage_heads(y4, hd_w, hd_bias):
    n = y4.shape[0] // 4
    nh = 64 if n % 64 == 0 else n
    return pl.pallas_call(
        _heads_kernel,
        out_shape=jax.ShapeDtypeStruct((n, 256), jnp.float32),
        grid=(n // nh,),
        in_specs=[pl.BlockSpec((nh * 4, 512), lambda i: (i, 0)),
                  pl.BlockSpec(hd_w.shape, _full(2)),
                  pl.BlockSpec(hd_bias.shape, _full(2))],
        out_specs=pl.BlockSpec((nh, 256), lambda i: (i, 0)),
        compiler_params=pltpu.CompilerParams(
            dimension_semantics=("parallel",)),
    )(y4, hd_w, hd_bias)


def kernel(x, c1_w, c1_scale, c1_bias, l0b0_c1_w, l0b0_c1_scale, l0b0_c1_bias, l0b0_c2_w, l0b0_c2_scale, l0b0_c2_bias, l0b1_c1_w, l0b1_c1_scale, l0b1_c1_bias, l0b1_c2_w, l0b1_c2_scale, l0b1_c2_bias, l1b0_c1_w, l1b0_c1_scale, l1b0_c1_bias, l1b0_c2_w, l1b0_c2_scale, l1b0_c2_bias, l1b0_dn_w, l1b0_dn_scale, l1b0_dn_bias, l1b1_c1_w, l1b1_c1_scale, l1b1_c1_bias, l1b1_c2_w, l1b1_c2_scale, l1b1_c2_bias, l2b0_c1_w, l2b0_c1_scale, l2b0_c1_bias, l2b0_c2_w, l2b0_c2_scale, l2b0_c2_bias, l2b0_dn_w, l2b0_dn_scale, l2b0_dn_bias, l2b1_c1_w, l2b1_c1_scale, l2b1_c1_bias, l2b1_c2_w, l2b1_c2_scale, l2b1_c2_bias, l3b0_c1_w, l3b0_c1_scale, l3b0_c1_bias, l3b0_c2_w, l3b0_c2_scale, l3b0_c2_bias, l3b0_dn_w, l3b0_dn_scale, l3b0_dn_bias, l3b1_c1_w, l3b1_c1_scale, l3b1_c1_bias, l3b1_c2_w, l3b1_c2_scale, l3b1_c2_bias, hd_w, hd_bias):
    n = x.shape[0]
    y = _stage_a(x, c1_w, c1_scale, c1_bias)

    cfg = [
        ((l0b0_c1_w, l0b0_c1_scale, l0b0_c1_bias),
         (l0b0_c2_w, l0b0_c2_scale, l0b0_c2_bias), None, 1),
        ((l0b1_c1_w, l0b1_c1_scale, l0b1_c1_bias),
         (l0b1_c2_w, l0b1_c2_scale, l0b1_c2_bias), None, 1),
        ((l1b0_c1_w, l1b0_c1_scale, l1b0_c1_bias),
         (l1b0_c2_w, l1b0_c2_scale, l1b0_c2_bias),
         (l1b0_dn_w, l1b0_dn_scale, l1b0_dn_bias), 2),
        ((l1b1_c1_w, l1b1_c1_scale, l1b1_c1_bias),
         (l1b1_c2_w, l1b1_c2_scale, l1b1_c2_bias), None, 1),
        ((l2b0_c1_w, l2b0_c1_scale, l2b0_c1_bias),
         (l2b0_c2_w, l2b0_c2_scale, l2b0_c2_bias),
         (l2b0_dn_w, l2b0_dn_scale, l2b0_dn_bias), 2),
        ((l2b1_c1_w, l2b1_c1_scale, l2b1_c1_bias),
         (l2b1_c2_w, l2b1_c2_scale, l2b1_c2_bias), None, 1),
        ((l3b0_c1_w, l3b0_c1_scale, l3b0_c1_bias),
         (l3b0_c2_w, l3b0_c2_scale, l3b0_c2_bias),
         (l3b0_dn_w, l3b0_dn_scale, l3b0_dn_bias), 2),
        ((l3b1_c1_w, l3b1_c1_scale, l3b1_c1_bias),
         (l3b1_c2_w, l3b1_c2_scale, l3b1_c2_bias), None, 1),
    ]
    h, c = 16, 128
    for c1, c2, dn, stride in cfg:
        y, h, c = _block(y, n, h, c, c1, c2, dn, stride)

    logits = _stage_heads(y, hd_w, hd_bias)
    return logits[:, :200]
